# baseline (speedup 1.0000x reference)
import functools

import jax
import jax.numpy as jnp
from jax import lax
from jax.experimental import pallas as pl
from jax.experimental.pallas import tpu as pltpu
from jax.experimental.pallas import tpu_sc as plsc

_N = 16384
_K = 50
_EMB = 64

_NC = 2
_NS = 16
_NW = _NC * _NS
_ROWS_W = _N // _NW
_CH_ROWS = 16
_KP = _K // 2
_CH_P = _CH_ROWS * _KP
_N_CH = _ROWS_W // _CH_ROWS
_GB = 80
_NG = _CH_P // _GB
_GRP = _GB // 16
_L = 16
_NP = _N * _K // 2


def _bucket(d):
    b = jnp.minimum(jnp.maximum(d - 1, 0), 1)
    for thr in (2, 3, 4, 7, 15, 31, 63):
        b = b + jnp.minimum(jnp.maximum(d - thr, 0), 1)
    return b


def _sc_body(tope_hbm, topo_hbm, pt_hbm, out_hbm,
             te_v, to_v, i_v, o_v0, o_v1, pt_sh, sem_g, sem_w0, sem_w1):
    wid = lax.axis_index("s") * _NC + lax.axis_index("c")
    lane = lax.iota(jnp.int32, _L)

    sid = lax.axis_index("s")
    pltpu.sync_copy(pt_hbm, pt_sh.at[pl.ds(sid * 88, 88)])
    plsc.subcore_barrier()
    repl_off = sid * 88

    def chunk_p0(c):
        return pl.multiple_of((wid * _ROWS_W + c * _CH_ROWS) * _KP, _CH_P)

    def produce(c, o_v):
        row0 = wid * _ROWS_W + c * _CH_ROWS
        p0 = chunk_p0(c)
        pltpu.sync_copy(tope_hbm.at[pl.ds(p0, _CH_P)], te_v)
        pltpu.sync_copy(topo_hbm.at[pl.ds(p0, _CH_P)], to_v)
        cps = []
        for gb in range(_NG):
            for g in range(gb * _GRP, (gb + 1) * _GRP):
                off = (g * _L) // _KP
                rem = (g * _L) % _KP
                i = row0 + off
                if rem + _L > _KP:
                    split = _KP - rem
                    i = i + jnp.minimum(jnp.maximum(lane - (split - 1), 0), 1)
                de = jnp.maximum(i - te_v[pl.ds(g * _L, _L)], 1)
                do = jnp.maximum(i - to_v[pl.ds(g * _L, _L)], 1)
                i_v[pl.ds(g * _L, _L)] = _bucket(de) * 9 + _bucket(do) + repl_off
            cps.append(pltpu.async_copy(
                pt_sh.at[i_v.at[pl.ds(gb * _GB, _GB)]],
                o_v.at[pl.ds(gb * _GB, _GB)],
                sem_g,
            ))
        for cp in cps:
            cp.wait()

    def wb_start(c, o_v, sem):
        return pltpu.async_copy(
            o_v, out_hbm.at[pl.ds(chunk_p0(c), _CH_P)], sem)

    produce(jnp.int32(0), o_v0)

    def step(c2, carry):
        wb0 = wb_start(2 * c2, o_v0, sem_w0)
        produce(2 * c2 + 1, o_v1)
        wb0.wait()
        wb1 = wb_start(2 * c2 + 1, o_v1, sem_w1)
        produce(2 * c2 + 2, o_v0)
        wb1.wait()
        return carry

    lax.fori_loop(0, (_N_CH - 2) // 2, step, 0)
    wb0 = wb_start(jnp.int32(_N_CH - 2), o_v0, sem_w0)
    produce(jnp.int32(_N_CH - 1), o_v1)
    wb0.wait()
    wb_start(jnp.int32(_N_CH - 1), o_v1, sem_w1).wait()


@functools.partial(jax.jit)
def _run(top_even, top_odd, pair_table):
    mesh = plsc.VectorSubcoreMesh(core_axis_name="c", subcore_axis_name="s")
    fn = pl.kernel(
        _sc_body,
        mesh=mesh,
        out_type=jax.ShapeDtypeStruct((_NP, 2 * _EMB), jnp.float32),
        scratch_types=[
            pltpu.VMEM((_CH_P,), jnp.int32),
            pltpu.VMEM((_CH_P,), jnp.int32),
            pltpu.VMEM((_CH_P,), jnp.int32),
            pltpu.VMEM((_CH_P, 2 * _EMB), jnp.float32),
            pltpu.VMEM((_CH_P, 2 * _EMB), jnp.float32),
            pltpu.VMEM_SHARED((16 * 88, 2 * _EMB), jnp.float32),
            pltpu.SemaphoreType.DMA,
            pltpu.SemaphoreType.DMA,
            pltpu.SemaphoreType.DMA,
        ],
    )
    return fn(top_even, top_odd, pair_table)


def kernel(top_indices, distance_emb):
    top2 = top_indices.reshape(_NP, 2)
    top_even = top2[:, 0]
    top_odd = top2[:, 1]
    ia, ib = jnp.divmod(jnp.arange(81, dtype=jnp.int32), 9)
    pair_table = jnp.concatenate(
        [distance_emb[ia], distance_emb[ib]], axis=-1)
    pair_table = jnp.pad(pair_table, ((0, 7), (0, 0)))
    out = _run(top_even, top_odd, pair_table)
    return out.reshape(_N, _K, _EMB)


# speedup vs baseline: 1.2578x; 1.2578x over previous
import functools

import jax
import jax.numpy as jnp
from jax import lax
from jax.experimental import pallas as pl
from jax.experimental.pallas import tpu as pltpu
from jax.experimental.pallas import tpu_sc as plsc

_N = 16384
_K = 50
_EMB = 64

_NC = 2
_NS = 16
_NW = _NC * _NS
_ROWS_W = _N // _NW
_E_W = _ROWS_W * _K
_M_ROWS = 8
_M_E = _M_ROWS * _K
_N_M = _ROWS_W // _M_ROWS
_L = 16

_TC_B = 6400
_NB = _N * _K // _TC_B


def _bucket(d):
    b = jnp.minimum(jnp.maximum(d - 1, 0), 1)
    for thr in (2, 3, 4, 7, 15, 31, 63):
        b = b + jnp.minimum(jnp.maximum(d - thr, 0), 1)
    return b


def _sc_index_body(top_hbm, idx_hbm, t_v):
    wid = lax.axis_index("s") * _NC + lax.axis_index("c")
    lane = lax.iota(jnp.int32, _L)
    e0 = wid * _E_W
    pltpu.sync_copy(top_hbm.at[pl.ds(e0, _E_W)], t_v)

    def macro(m, carry):
        row0 = wid * _ROWS_W + m * _M_ROWS
        base = m * _M_E
        for g in range(_M_E // _L):
            off = (g * _L) // _K
            rem = (g * _L) % _K
            i = row0 + off
            if rem + _L > _K:
                split = _K - rem
                i = i + jnp.minimum(jnp.maximum(lane - (split - 1), 0), 1)
            t = t_v[pl.ds(base + g * _L, _L)]
            d = jnp.maximum(i - t, 1)
            t_v[pl.ds(base + g * _L, _L)] = _bucket(d)
        return carry

    lax.fori_loop(0, _N_M, macro, 0)
    pltpu.sync_copy(t_v, idx_hbm.at[pl.ds(e0, _E_W)])


def _tc_expand_body(idx_ref, tab_ref, out_ref):
    b = idx_ref[0, 0, :]
    oh = (b[:, None] == lax.broadcasted_iota(jnp.int32, (1, 16), 1))
    out_ref[...] = jnp.dot(oh.astype(jnp.float32), tab_ref[...],
                           preferred_element_type=jnp.float32)


@functools.partial(jax.jit)
def _run(top_flat, distance_emb):
    mesh = plsc.VectorSubcoreMesh(core_axis_name="c", subcore_axis_name="s")
    sc_index = pl.kernel(
        _sc_index_body,
        mesh=mesh,
        out_type=jax.ShapeDtypeStruct((_N * _K,), jnp.int32),
        scratch_types=[pltpu.VMEM((_E_W,), jnp.int32)],
    )
    idx = sc_index(top_flat)
    tab16 = jnp.zeros((16, _EMB), jnp.float32).at[:9].set(distance_emb)
    out = pl.pallas_call(
        _tc_expand_body,
        grid=(_NB,),
        in_specs=[
            pl.BlockSpec((1, 1, _TC_B), lambda i: (i, 0, 0)),
            pl.BlockSpec((16, _EMB), lambda i: (0, 0)),
        ],
        out_specs=pl.BlockSpec((_TC_B, _EMB), lambda i: (i, 0)),
        out_shape=jax.ShapeDtypeStruct((_N * _K, _EMB), jnp.float32),
    )(idx.reshape(_NB, 1, _TC_B), tab16)
    return out


def kernel(top_indices, distance_emb):
    out = _run(top_indices.reshape(-1), distance_emb)
    return out.reshape(_N, _K, _EMB)


# speedup vs baseline: 1.3245x; 1.0530x over previous
import functools

import jax
import jax.numpy as jnp
from jax import lax
from jax.experimental import pallas as pl
from jax.experimental.pallas import tpu as pltpu
from jax.experimental.pallas import tpu_sc as plsc

_N = 16384
_K = 50
_EMB = 64

_NC = 2
_NS = 16
_NW = _NC * _NS
_ROWS_W = _N // _NW
_E_W = _ROWS_W * _K
_M_ROWS = 8
_M_E = _M_ROWS * _K
_N_M = _ROWS_W // _M_ROWS
_L = 16

_TC_B = 25600
_NB = _N * _K // _TC_B


def _bucket(d):
    b = jnp.minimum(jnp.maximum(d - 1, 0), 1)
    for thr in (2, 3, 4, 7, 15, 31, 63):
        b = b + jnp.minimum(jnp.maximum(d - thr, 0), 1)
    return b


def _sc_index_body(top_hbm, idx_hbm, t_v):
    wid = lax.axis_index("s") * _NC + lax.axis_index("c")
    lane = lax.iota(jnp.int32, _L)
    e0 = wid * _E_W
    pltpu.sync_copy(top_hbm.at[pl.ds(e0, _E_W)], t_v)

    def macro(m, carry):
        row0 = wid * _ROWS_W + m * _M_ROWS
        base = m * _M_E
        for g in range(_M_E // _L):
            off = (g * _L) // _K
            rem = (g * _L) % _K
            i = row0 + off
            if rem + _L > _K:
                split = _K - rem
                i = i + jnp.minimum(jnp.maximum(lane - (split - 1), 0), 1)
            t = t_v[pl.ds(base + g * _L, _L)]
            d = jnp.maximum(i - t, 1)
            t_v[pl.ds(base + g * _L, _L)] = _bucket(d)
        return carry

    lax.fori_loop(0, _N_M, macro, 0)
    pltpu.sync_copy(t_v, idx_hbm.at[pl.ds(e0, _E_W)])


def _tc_expand_body(idx_ref, tab_ref, out_ref):
    b = idx_ref[0, 0, :]
    oh = (b[:, None] == lax.broadcasted_iota(jnp.int32, (1, 16), 1))
    out_ref[...] = jnp.dot(oh.astype(jnp.float32), tab_ref[...],
                           preferred_element_type=jnp.float32)


@functools.partial(jax.jit)
def _run(top_flat, distance_emb):
    mesh = plsc.VectorSubcoreMesh(core_axis_name="c", subcore_axis_name="s")
    sc_index = pl.kernel(
        _sc_index_body,
        mesh=mesh,
        out_type=jax.ShapeDtypeStruct((_N * _K,), jnp.int32),
        scratch_types=[pltpu.VMEM((_E_W,), jnp.int32)],
    )
    idx = sc_index(top_flat)
    tab16 = jnp.zeros((16, _EMB), jnp.float32).at[:9].set(distance_emb)
    out = pl.pallas_call(
        _tc_expand_body,
        grid=(_NB,),
        in_specs=[
            pl.BlockSpec((1, 1, _TC_B), lambda i: (i, 0, 0)),
            pl.BlockSpec((16, _EMB), lambda i: (0, 0)),
        ],
        out_specs=pl.BlockSpec((_TC_B, _EMB), lambda i: (i, 0)),
        out_shape=jax.ShapeDtypeStruct((_N * _K, _EMB), jnp.float32),
    )(idx.reshape(_NB, 1, _TC_B), tab16)
    return out


def kernel(top_indices, distance_emb):
    out = _run(top_indices.reshape(-1), distance_emb)
    return out.reshape(_N, _K, _EMB)


# speedup vs baseline: 1.3313x; 1.0051x over previous
import functools

import jax
import jax.numpy as jnp
from jax import lax
from jax.experimental import pallas as pl
from jax.experimental.pallas import tpu as pltpu
from jax.experimental.pallas import tpu_sc as plsc

_N = 16384
_K = 50
_EMB = 64

_NC = 2
_NS = 16
_NW = _NC * _NS
_ROWS_W = _N // _NW
_E_W = _ROWS_W * _K
_M_ROWS = 8
_M_E = _M_ROWS * _K
_N_M = _ROWS_W // _M_ROWS
_L = 16

_TC_B = 25600
_NB = _N * _K // _TC_B


def _bucket(d):
    b = jnp.minimum(jnp.maximum(d - 1, 0), 1)
    for thr in (2, 3, 4, 7, 15, 31, 63):
        b = b + jnp.minimum(jnp.maximum(d - thr, 0), 1)
    return b


def _sc_index_body(top_hbm, idx_hbm, t2_v, b_v):
    wid = lax.axis_index("s") * _NC + lax.axis_index("c")
    r0 = wid * _ROWS_W
    pltpu.sync_copy(top_hbm.at[pl.ds(r0, _ROWS_W), :], t2_v)

    def macro(m, carry):
        for r in range(_M_ROWS):
            i = r0 + m * _M_ROWS + r
            for off in (0, 16, 32, _K - _L):
                t = t2_v[m * _M_ROWS + r, pl.ds(off, _L)]
                d = jnp.maximum(i - t, 1)
                b_v[pl.ds((m * _M_ROWS + r) * _K + off, _L)] = _bucket(d)
        return carry

    lax.fori_loop(0, _N_M, macro, 0)
    pltpu.sync_copy(b_v, idx_hbm.at[pl.ds(r0 * _K, _E_W)])


def _tc_expand_body(idx_ref, tab_ref, out_ref):
    b = idx_ref[0, 0, :]
    oh = (b[:, None] == lax.broadcasted_iota(jnp.int32, (1, 16), 1))
    out_ref[...] = jnp.dot(oh.astype(jnp.float32), tab_ref[...],
                           preferred_element_type=jnp.float32)


@functools.partial(jax.jit)
def _run(top_2d, distance_emb):
    mesh = plsc.VectorSubcoreMesh(core_axis_name="c", subcore_axis_name="s")
    sc_index = pl.kernel(
        _sc_index_body,
        mesh=mesh,
        out_type=jax.ShapeDtypeStruct((_N * _K,), jnp.int32),
        scratch_types=[
            pltpu.VMEM((_ROWS_W, _K), jnp.int32),
            pltpu.VMEM((_E_W,), jnp.int32),
        ],
    )
    idx = sc_index(top_2d)
    tab16 = jnp.zeros((16, _EMB), jnp.float32).at[:9].set(distance_emb)
    out = pl.pallas_call(
        _tc_expand_body,
        grid=(_NB,),
        in_specs=[
            pl.BlockSpec((1, 1, _TC_B), lambda i: (i, 0, 0)),
            pl.BlockSpec((16, _EMB), lambda i: (0, 0)),
        ],
        out_specs=pl.BlockSpec((_TC_B, _EMB), lambda i: (i, 0)),
        out_shape=jax.ShapeDtypeStruct((_N * _K, _EMB), jnp.float32),
    )(idx.reshape(_NB, 1, _TC_B), tab16)
    return out


def kernel(top_indices, distance_emb):
    out = _run(top_indices, distance_emb)
    return out.reshape(_N, _K, _EMB)
